# baseline (speedup 1.0000x reference)
import functools

import jax
import jax.numpy as jnp
from jax import lax
from jax.experimental import pallas as pl
from jax.experimental.pallas import tpu as pltpu
from jax.experimental.pallas import tpu_sc as plsc

B = 16384
N_DENSE = 13
N_SPARSE = 26
VOCAB = 100000
EMB_DIM = 16

NC = 2
NS = 16
NW = NC * NS

BPT = B // NW
RPT = BPT * N_SPARSE
NROWS = N_SPARSE * VOCAB
TN = 104
NT = RPT // TN
SGB = 16
NSG = BPT // SGB
OFF_PERIOD = 208
OFF_VECS = OFF_PERIOD // 16


def _sc_body(rs_hbm, idx_hbm, offs_hbm, dvt_hbm, dwt_hbm, out_hbm,
             idx_v, offs_v, dvt_v, dwt_v, out_v, val_v, sem):
    wid = lax.axis_index("s") * NC + lax.axis_index("c")
    ibase = wid * RPT
    bbase = wid * BPT

    pltpu.sync_copy(idx_hbm.at[pl.ds(ibase, RPT)], idx_v)
    pltpu.sync_copy(offs_hbm, offs_v)
    pltpu.sync_copy(dwt_hbm, dwt_v)
    pltpu.sync_copy(dvt_hbm.at[:, pl.ds(bbase, BPT)], dvt_v)

    def off_body(o, carry):
        for j in range(OFF_VECS):
            sl = pl.ds((o * OFF_VECS + j) * 16, 16)
            idx_v[sl] = idx_v[sl] + offs_v[pl.ds(j * 16, 16)]
        return carry

    lax.fori_loop(0, RPT // OFF_PERIOD, off_body, 0)

    def fire_body(t, carry):
        pltpu.make_async_copy(
            rs_hbm.at[idx_v.at[pl.ds(t * TN, TN)]],
            val_v.at[pl.ds(t * TN, TN)], sem).start()
        return carry

    lax.fori_loop(0, NT, fire_body, 0)

    pltpu.make_async_copy(rs_hbm.at[pl.ds(0, RPT)], val_v, sem).wait()

    lanes26 = lax.iota(jnp.int32, 16) * N_SPARSE

    def sg_body(sg, carry):
        base = lanes26 + sg * (SGB * N_SPARSE)
        b0 = plsc.load_gather(val_v, [base + 0])
        b1 = plsc.load_gather(val_v, [base + 1])
        b2 = dvt_v[0, pl.ds(sg * SGB, 16)] * dwt_v[0, :]
        b3 = dvt_v[1, pl.ds(sg * SGB, 16)] * dwt_v[1, :]
        for f in range(2, N_SPARSE, 2):
            b0 = b0 + plsc.load_gather(val_v, [base + f])
            b1 = b1 + plsc.load_gather(val_v, [base + f + 1])
        for k in range(2, EMB_DIM, 2):
            b2 = b2 + dvt_v[k, pl.ds(sg * SGB, 16)] * dwt_v[k, :]
            b3 = b3 + dvt_v[k + 1, pl.ds(sg * SGB, 16)] * dwt_v[k + 1, :]
        out_v[pl.ds(sg * SGB, 16)] = (b0 + b1) + (b2 + b3)
        return carry

    lax.fori_loop(0, NSG, sg_body, 0)

    pltpu.sync_copy(out_v, out_hbm.at[pl.ds(bbase, BPT)])


@functools.partial(
    pl.kernel,
    out_type=jax.ShapeDtypeStruct((B,), jnp.float32),
    mesh=plsc.VectorSubcoreMesh(core_axis_name="c", subcore_axis_name="s"),
    compiler_params=pltpu.CompilerParams(needs_layout_passes=False),
    scratch_types=[
        pltpu.VMEM((RPT,), jnp.int32),
        pltpu.VMEM((OFF_PERIOD,), jnp.int32),
        pltpu.VMEM((EMB_DIM, BPT), jnp.float32),
        pltpu.VMEM((EMB_DIM, EMB_DIM), jnp.float32),
        pltpu.VMEM((BPT,), jnp.float32),
        pltpu.VMEM((RPT,), jnp.float32),
        pltpu.SemaphoreType.DMA,
    ],
)
def _sc_linear(rs_hbm, idx_hbm, offs_hbm, dvt_hbm, dwt_hbm, out_hbm,
               idx_v, offs_v, dvt_v, dwt_v, out_v, val_v, sem):
    _sc_body(rs_hbm, idx_hbm, offs_hbm, dvt_hbm, dwt_hbm, out_hbm,
             idx_v, offs_v, dvt_v, dwt_v, out_v, val_v, sem)


DM_ROWS = N_SPARSE * EMB_DIM
VB_FULL = VOCAB // 128
NBLK = N_SPARSE * VB_FULL
TAIL_V = VB_FULL * 128
BPW = (NBLK + NW - 1) // NW


def _rowsum_body(dm_hbm, rs_hbm, blk0, blk1, outb, tailb, outt, sem0, sem1):
    blks = (blk0, blk1)
    sems = (sem0, sem1)
    wid = lax.axis_index("s") * NC + lax.axis_index("c")

    def fire(i, slot):
        bid = wid + i * NW

        @pl.when(bid < NBLK)
        def _():
            f = bid // VB_FULL
            vb = bid - f * VB_FULL
            pltpu.make_async_copy(
                dm_hbm.at[pl.ds(f * EMB_DIM, EMB_DIM), pl.ds(vb * 128, 128)],
                blks[slot], sems[slot]).start()

    def drain(i, slot):
        bid = wid + i * NW

        @pl.when(bid < NBLK)
        def _():
            f = bid // VB_FULL
            vb = bid - f * VB_FULL
            pltpu.make_async_copy(
                dm_hbm.at[pl.ds(f * EMB_DIM, EMB_DIM), pl.ds(vb * 128, 128)],
                blks[slot], sems[slot]).wait()

    for s in range(2):
        fire(s, s)

    def blk_body(g, carry):
        for s in range(2):
            i = g * 2 + s
            bid = wid + i * NW
            drain(i, s)

            @pl.when(bid < NBLK)
            def _():
                blk = blks[s]
                f = bid // VB_FULL
                vb = bid - f * VB_FULL
                for q in range(8):
                    acc = blk[0, pl.ds(q * 16, 16)] + blk[1, pl.ds(q * 16, 16)]
                    a1 = blk[2, pl.ds(q * 16, 16)] + blk[3, pl.ds(q * 16, 16)]
                    for rr in range(4, EMB_DIM, 2):
                        acc = acc + blk[rr, pl.ds(q * 16, 16)]
                        a1 = a1 + blk[rr + 1, pl.ds(q * 16, 16)]
                    outb[pl.ds(q * 16, 16)] = acc + a1
                pltpu.sync_copy(
                    outb, rs_hbm.at[pl.ds(f * VOCAB + vb * 128, 128)])

            fire(i + 2, s)
        return carry

    lax.fori_loop(0, (BPW + 1) // 2, blk_body, 0)

    @pl.when(wid < N_SPARSE)
    def _():
        pltpu.sync_copy(
            dm_hbm.at[pl.ds(wid * EMB_DIM, EMB_DIM), pl.ds(TAIL_V, 32)],
            tailb)
        for q in range(2):
            acc = tailb[0, pl.ds(q * 16, 16)] + tailb[1, pl.ds(q * 16, 16)]
            a1 = tailb[2, pl.ds(q * 16, 16)] + tailb[3, pl.ds(q * 16, 16)]
            for rr in range(4, EMB_DIM, 2):
                acc = acc + tailb[rr, pl.ds(q * 16, 16)]
                a1 = a1 + tailb[rr + 1, pl.ds(q * 16, 16)]
            outt[pl.ds(q * 16, 16)] = acc + a1
        pltpu.sync_copy(outt, rs_hbm.at[pl.ds(wid * VOCAB + TAIL_V, 32)])


@functools.partial(
    pl.kernel,
    out_type=jax.ShapeDtypeStruct((NROWS,), jnp.float32),
    mesh=plsc.VectorSubcoreMesh(core_axis_name="c", subcore_axis_name="s"),
    compiler_params=pltpu.CompilerParams(needs_layout_passes=False),
    scratch_types=[
        pltpu.VMEM((EMB_DIM, 128), jnp.float32),
        pltpu.VMEM((EMB_DIM, 128), jnp.float32),
        pltpu.VMEM((128,), jnp.float32),
        pltpu.VMEM((EMB_DIM, 32), jnp.float32),
        pltpu.VMEM((32,), jnp.float32),
        pltpu.SemaphoreType.DMA,
        pltpu.SemaphoreType.DMA,
    ],
)
def _sc_rowsum(dm_hbm, rs_hbm, blk0, blk1, outb, tailb, outt, sem0, sem1):
    _rowsum_body(dm_hbm, rs_hbm, blk0, blk1, outb, tailb, outt, sem0, sem1)


def kernel(inputs, emb_tables, dense_weight, bias):
    idx = inputs[:, N_DENSE:N_DENSE + N_SPARSE].astype(jnp.int32).reshape(-1)
    dm = jnp.transpose(emb_tables, (0, 2, 1)).reshape(
        N_SPARSE * EMB_DIM, VOCAB)
    rowsum = _sc_rowsum(dm)
    offs = ((jnp.arange(OFF_PERIOD, dtype=jnp.int32) % N_SPARSE)
            * jnp.int32(VOCAB))
    dvt = jnp.concatenate(
        [inputs[:, :N_DENSE],
         jnp.ones((B, 1), jnp.float32),
         jnp.zeros((B, EMB_DIM - N_DENSE - 1), jnp.float32)], axis=1).T
    dwt = jnp.tile(
        jnp.concatenate([dense_weight[:, 0], bias,
                         jnp.zeros((EMB_DIM - N_DENSE - 1,), jnp.float32)]
                        )[:, None], (1, EMB_DIM))
    out = _sc_linear(rowsum, idx, offs, dvt, dwt)
    return out.reshape(B, 1)


# speedup vs baseline: 1.6273x; 1.6273x over previous
import functools

import jax
import jax.numpy as jnp
from jax import lax
from jax.experimental import pallas as pl
from jax.experimental.pallas import tpu as pltpu
from jax.experimental.pallas import tpu_sc as plsc

B = 16384
N_DENSE = 13
N_SPARSE = 26
VOCAB = 100000
EMB_DIM = 16

NC = 2
NS = 16
NW = NC * NS

BPT = B // NW
RPT = BPT * N_SPARSE
NROWS = N_SPARSE * VOCAB
TN = 104
NT = RPT // TN
SGB = 16
NSG = BPT // SGB
OFF_PERIOD = 208
OFF_VECS = OFF_PERIOD // 16


def _sc_body(rs_hbm, idx_hbm, offs_hbm, dvt_hbm, dwt_hbm, out_hbm,
             idx_v, offs_v, dvt_v, dwt_v, out_v, val_v, sem):
    wid = lax.axis_index("s") * NC + lax.axis_index("c")
    ibase = wid * RPT
    bbase = wid * BPT

    pltpu.sync_copy(idx_hbm.at[pl.ds(ibase, RPT)], idx_v)
    pltpu.sync_copy(offs_hbm, offs_v)
    pltpu.sync_copy(dwt_hbm, dwt_v)
    pltpu.sync_copy(dvt_hbm.at[:, pl.ds(bbase, BPT)], dvt_v)

    def off_body(o, carry):
        for j in range(OFF_VECS):
            sl = pl.ds((o * OFF_VECS + j) * 16, 16)
            idx_v[sl] = idx_v[sl] + offs_v[pl.ds(j * 16, 16)]
        return carry

    lax.fori_loop(0, RPT // OFF_PERIOD, off_body, 0)

    def fire_body(t, carry):
        pltpu.make_async_copy(
            rs_hbm.at[idx_v.at[pl.ds(t * TN, TN)]],
            val_v.at[pl.ds(t * TN, TN)], sem).start()
        return carry

    lax.fori_loop(0, NT, fire_body, 0)

    pltpu.make_async_copy(rs_hbm.at[pl.ds(0, RPT)], val_v, sem).wait()

    lanes26 = lax.iota(jnp.int32, 16) * N_SPARSE

    def sg_body(sg, carry):
        base = lanes26 + sg * (SGB * N_SPARSE)
        b0 = plsc.load_gather(val_v, [base + 0])
        b1 = plsc.load_gather(val_v, [base + 1])
        b2 = dvt_v[0, pl.ds(sg * SGB, 16)] * dwt_v[0, :]
        b3 = dvt_v[1, pl.ds(sg * SGB, 16)] * dwt_v[1, :]
        for f in range(2, N_SPARSE, 2):
            b0 = b0 + plsc.load_gather(val_v, [base + f])
            b1 = b1 + plsc.load_gather(val_v, [base + f + 1])
        for k in range(2, EMB_DIM, 2):
            b2 = b2 + dvt_v[k, pl.ds(sg * SGB, 16)] * dwt_v[k, :]
            b3 = b3 + dvt_v[k + 1, pl.ds(sg * SGB, 16)] * dwt_v[k + 1, :]
        out_v[pl.ds(sg * SGB, 16)] = (b0 + b1) + (b2 + b3)
        return carry

    lax.fori_loop(0, NSG, sg_body, 0)

    pltpu.sync_copy(out_v, out_hbm.at[pl.ds(bbase, BPT)])


@functools.partial(
    pl.kernel,
    out_type=jax.ShapeDtypeStruct((B,), jnp.float32),
    mesh=plsc.VectorSubcoreMesh(core_axis_name="c", subcore_axis_name="s"),
    compiler_params=pltpu.CompilerParams(needs_layout_passes=False),
    scratch_types=[
        pltpu.VMEM((RPT,), jnp.int32),
        pltpu.VMEM((OFF_PERIOD,), jnp.int32),
        pltpu.VMEM((EMB_DIM, BPT), jnp.float32),
        pltpu.VMEM((EMB_DIM, EMB_DIM), jnp.float32),
        pltpu.VMEM((BPT,), jnp.float32),
        pltpu.VMEM((RPT,), jnp.float32),
        pltpu.SemaphoreType.DMA,
    ],
)
def _sc_linear(rs_hbm, idx_hbm, offs_hbm, dvt_hbm, dwt_hbm, out_hbm,
               idx_v, offs_v, dvt_v, dwt_v, out_v, val_v, sem):
    _sc_body(rs_hbm, idx_hbm, offs_hbm, dvt_hbm, dwt_hbm, out_hbm,
             idx_v, offs_v, dvt_v, dwt_v, out_v, val_v, sem)


DM_ROWS = N_SPARSE * EMB_DIM
W_COLS = 3072
T31_BASE = 31 * W_COLS
T31_COLS = VOCAB - VOCAB % 128 - T31_BASE
TAIL_V = VOCAB - VOCAB % 128
SUBC = 768
SUBC31 = 1152


def _rowsum_body(dm_hbm, tail_hbm, rs_hbm, blk0, blk1, outf, tailb, outt,
                 sem0, sem1):
    blks = (blk0, blk1)
    sems = (sem0, sem1)
    wid = lax.axis_index("s") * NC + lax.axis_index("c")

    def reduce_cols(buf, ncols, obase):
        def qbody(q, carry):
            acc = buf[0, pl.ds(q * 16, 16)] + buf[1, pl.ds(q * 16, 16)]
            a1 = buf[2, pl.ds(q * 16, 16)] + buf[3, pl.ds(q * 16, 16)]
            for rr in range(4, EMB_DIM, 2):
                acc = acc + buf[rr, pl.ds(q * 16, 16)]
                a1 = a1 + buf[rr + 1, pl.ds(q * 16, 16)]
            outf[pl.ds(obase + q * 16, 16)] = acc + a1
            return carry
        lax.fori_loop(0, ncols // 16, qbody, 0)

    def mk_ring(cbase, subc):
        def fire(f, sub):
            pltpu.make_async_copy(
                dm_hbm.at[pl.ds(f * EMB_DIM, EMB_DIM),
                          pl.ds(cbase + sub * subc, subc)],
                blks[sub % 2].at[:, pl.ds(0, subc)], sems[sub % 2]).start()

        def drain(f, sub):
            pltpu.make_async_copy(
                dm_hbm.at[pl.ds(f * EMB_DIM, EMB_DIM),
                          pl.ds(cbase + sub * subc, subc)],
                blks[sub % 2].at[:, pl.ds(0, subc)], sems[sub % 2]).wait()
        return fire, drain

    @pl.when(wid < 31)
    def _():
        cbase = wid * W_COLS
        fire, drain = mk_ring(cbase, SUBC)
        fire(0, 0)
        fire(0, 1)

        def f_body(f, carry):
            for sub in range(4):
                drain(f, sub)
                reduce_cols(blks[sub % 2], SUBC, sub * SUBC)
                if sub < 2:
                    fire(f, sub + 2)
                else:
                    @pl.when(f < N_SPARSE - 1)
                    def _():
                        fire(f + 1, sub - 2)
            pltpu.sync_copy(outf.at[pl.ds(0, W_COLS)],
                            rs_hbm.at[pl.ds(f * VOCAB + cbase, W_COLS)])
            return carry

        lax.fori_loop(0, N_SPARSE, f_body, 0)

    @pl.when(wid == 31)
    def _():
        fire, drain = mk_ring(T31_BASE, SUBC31)
        fire(0, 0)
        fire(0, 1)

        def f_body(f, carry):
            for sub in range(4):
                drain(f, sub)
                reduce_cols(blks[sub % 2], SUBC31, sub * SUBC31)
                if sub < 2:
                    fire(f, sub + 2)
                else:
                    @pl.when(f < N_SPARSE - 1)
                    def _():
                        fire(f + 1, sub - 2)
            pltpu.sync_copy(
                dm_hbm.at[pl.ds(f * EMB_DIM, EMB_DIM),
                          pl.ds(T31_BASE + 4 * SUBC31, 128)],
                tailb.at[:, pl.ds(0, 128)])
            reduce_cols(tailb, 128, 4 * SUBC31)
            pltpu.sync_copy(outf.at[pl.ds(0, T31_COLS)],
                            rs_hbm.at[pl.ds(f * VOCAB + T31_BASE, T31_COLS)])
            return carry

        lax.fori_loop(0, N_SPARSE, f_body, 0)

    @pl.when(wid < N_SPARSE)
    def _():
        pltpu.sync_copy(tail_hbm.at[pl.ds(wid * 32, 32)], outt)
        pltpu.sync_copy(outt, rs_hbm.at[pl.ds(wid * VOCAB + TAIL_V, 32)])


@functools.partial(
    pl.kernel,
    out_type=jax.ShapeDtypeStruct((NROWS,), jnp.float32),
    mesh=plsc.VectorSubcoreMesh(core_axis_name="c", subcore_axis_name="s"),
    compiler_params=pltpu.CompilerParams(needs_layout_passes=False),
    scratch_types=[
        pltpu.VMEM((EMB_DIM, SUBC31), jnp.float32),
        pltpu.VMEM((EMB_DIM, SUBC31), jnp.float32),
        pltpu.VMEM((T31_COLS,), jnp.float32),
        pltpu.VMEM((EMB_DIM, 128), jnp.float32),
        pltpu.VMEM((32,), jnp.float32),
        pltpu.SemaphoreType.DMA,
        pltpu.SemaphoreType.DMA,
    ],
)
def _sc_rowsum(dm_hbm, tail_hbm, rs_hbm, blk0, blk1, outf, tailb, outt,
               sem0, sem1):
    _rowsum_body(dm_hbm, tail_hbm, rs_hbm, blk0, blk1, outf, tailb, outt,
                 sem0, sem1)


def kernel(inputs, emb_tables, dense_weight, bias):
    idx = inputs[:, N_DENSE:N_DENSE + N_SPARSE].astype(jnp.int32).reshape(-1)
    dm = jnp.transpose(emb_tables, (0, 2, 1)).reshape(
        N_SPARSE * EMB_DIM, VOCAB)
    tail = jnp.sum(emb_tables[:, TAIL_V:, :], axis=2).reshape(-1)
    rowsum = _sc_rowsum(dm, tail)
    offs = ((jnp.arange(OFF_PERIOD, dtype=jnp.int32) % N_SPARSE)
            * jnp.int32(VOCAB))
    dvt = jnp.concatenate(
        [inputs[:, :N_DENSE],
         jnp.ones((B, 1), jnp.float32),
         jnp.zeros((B, EMB_DIM - N_DENSE - 1), jnp.float32)], axis=1).T
    dwt = jnp.tile(
        jnp.concatenate([dense_weight[:, 0], bias,
                         jnp.zeros((EMB_DIM - N_DENSE - 1,), jnp.float32)]
                        )[:, None], (1, EMB_DIM))
    out = _sc_linear(rowsum, idx, offs, dvt, dwt)
    return out.reshape(B, 1)


# speedup vs baseline: 1.6694x; 1.0259x over previous
import functools

import jax
import jax.numpy as jnp
from jax import lax
from jax.experimental import pallas as pl
from jax.experimental.pallas import tpu as pltpu
from jax.experimental.pallas import tpu_sc as plsc

B = 16384
N_DENSE = 13
N_SPARSE = 26
VOCAB = 100000
EMB_DIM = 16

NC = 2
NS = 16
NW = NC * NS

BPT = B // NW
RPT = BPT * N_SPARSE
NROWS = N_SPARSE * VOCAB
TN = 104
NT = RPT // TN
SGB = 16
NSG = BPT // SGB
OFF_PERIOD = 208
OFF_VECS = OFF_PERIOD // 16


def _sc_body(rs_hbm, idx_hbm, offs_hbm, dvt_hbm, dwt_hbm, out_hbm,
             idx_v, offs_v, dvt_v, dwt_v, out_v, val_v, sem):
    wid = lax.axis_index("s") * NC + lax.axis_index("c")
    ibase = wid * RPT
    bbase = wid * BPT

    pltpu.sync_copy(idx_hbm.at[pl.ds(ibase, RPT)], idx_v)
    pltpu.sync_copy(offs_hbm, offs_v)
    pltpu.sync_copy(dwt_hbm, dwt_v)
    pltpu.sync_copy(dvt_hbm.at[:, pl.ds(bbase, BPT)], dvt_v)

    def off_body(o, carry):
        for j in range(OFF_VECS):
            sl = pl.ds((o * OFF_VECS + j) * 16, 16)
            idx_v[sl] = idx_v[sl] + offs_v[pl.ds(j * 16, 16)]
        return carry

    lax.fori_loop(0, RPT // OFF_PERIOD, off_body, 0)

    def fire_body(t, carry):
        pltpu.make_async_copy(
            rs_hbm.at[idx_v.at[pl.ds(t * TN, TN)]],
            val_v.at[pl.ds(t * TN, TN)], sem).start()
        return carry

    lax.fori_loop(0, NT, fire_body, 0)

    pltpu.make_async_copy(rs_hbm.at[pl.ds(0, RPT)], val_v, sem).wait()

    lanes26 = lax.iota(jnp.int32, 16) * N_SPARSE

    def sg_body(sg, carry):
        base = lanes26 + sg * (SGB * N_SPARSE)
        b0 = plsc.load_gather(val_v, [base + 0])
        b1 = plsc.load_gather(val_v, [base + 1])
        b2 = dvt_v[0, pl.ds(sg * SGB, 16)] * dwt_v[0, :]
        b3 = dvt_v[1, pl.ds(sg * SGB, 16)] * dwt_v[1, :]
        for f in range(2, N_SPARSE, 2):
            b0 = b0 + plsc.load_gather(val_v, [base + f])
            b1 = b1 + plsc.load_gather(val_v, [base + f + 1])
        for k in range(2, EMB_DIM, 2):
            b2 = b2 + dvt_v[k, pl.ds(sg * SGB, 16)] * dwt_v[k, :]
            b3 = b3 + dvt_v[k + 1, pl.ds(sg * SGB, 16)] * dwt_v[k + 1, :]
        out_v[pl.ds(sg * SGB, 16)] = (b0 + b1) + (b2 + b3)
        return carry

    lax.fori_loop(0, NSG, sg_body, 0)

    pltpu.sync_copy(out_v, out_hbm.at[pl.ds(bbase, BPT)])


@functools.partial(
    pl.kernel,
    out_type=jax.ShapeDtypeStruct((B,), jnp.float32),
    mesh=plsc.VectorSubcoreMesh(core_axis_name="c", subcore_axis_name="s"),
    compiler_params=pltpu.CompilerParams(needs_layout_passes=False),
    scratch_types=[
        pltpu.VMEM((RPT,), jnp.int32),
        pltpu.VMEM((OFF_PERIOD,), jnp.int32),
        pltpu.VMEM((EMB_DIM, BPT), jnp.float32),
        pltpu.VMEM((EMB_DIM, EMB_DIM), jnp.float32),
        pltpu.VMEM((BPT,), jnp.float32),
        pltpu.VMEM((RPT,), jnp.float32),
        pltpu.SemaphoreType.DMA,
    ],
)
def _sc_linear(rs_hbm, idx_hbm, offs_hbm, dvt_hbm, dwt_hbm, out_hbm,
               idx_v, offs_v, dvt_v, dwt_v, out_v, val_v, sem):
    _sc_body(rs_hbm, idx_hbm, offs_hbm, dvt_hbm, dwt_hbm, out_hbm,
             idx_v, offs_v, dvt_v, dwt_v, out_v, val_v, sem)


DM_ROWS = N_SPARSE * EMB_DIM
W_COLS = 3072
T31_BASE = 31 * W_COLS
T31_COLS = VOCAB - VOCAB % 128 - T31_BASE
TAIL_V = VOCAB - VOCAB % 128
SUBC = 768
SUBC31 = 1152


def _rowsum_body(dm_hbm, tail_hbm, rs_hbm, blk0, blk1, blk2, blk3,
                 outf, tailb, outt, sem0, sem1, sem2, sem3):
    blks = (blk0, blk1, blk2, blk3)
    sems = (sem0, sem1, sem2, sem3)
    wid = lax.axis_index("s") * NC + lax.axis_index("c")

    def reduce_cols(buf, ncols, obase):
        def qbody(q, carry):
            acc = buf[0, pl.ds(q * 16, 16)] + buf[1, pl.ds(q * 16, 16)]
            a1 = buf[2, pl.ds(q * 16, 16)] + buf[3, pl.ds(q * 16, 16)]
            for rr in range(4, EMB_DIM, 2):
                acc = acc + buf[rr, pl.ds(q * 16, 16)]
                a1 = a1 + buf[rr + 1, pl.ds(q * 16, 16)]
            outf[pl.ds(obase + q * 16, 16)] = acc + a1
            return carry
        lax.fori_loop(0, ncols // 16, qbody, 0, unroll=2)

    def mk_ring(cbase, subc):
        def fire(f, sub):
            pltpu.make_async_copy(
                dm_hbm.at[pl.ds(f * EMB_DIM, EMB_DIM),
                          pl.ds(cbase + sub * subc, subc)],
                blks[sub].at[:, pl.ds(0, subc)], sems[sub]).start()

        def drain(f, sub):
            pltpu.make_async_copy(
                dm_hbm.at[pl.ds(f * EMB_DIM, EMB_DIM),
                          pl.ds(cbase + sub * subc, subc)],
                blks[sub].at[:, pl.ds(0, subc)], sems[sub]).wait()
        return fire, drain

    @pl.when(wid < 31)
    def _():
        cbase = wid * W_COLS
        fire, drain = mk_ring(cbase, SUBC)
        for sub in range(4):
            fire(0, sub)

        def f_body(f, carry):
            for sub in range(4):
                drain(f, sub)
                reduce_cols(blks[sub], SUBC, sub * SUBC)

                @pl.when(f < N_SPARSE - 1)
                def _():
                    fire(f + 1, sub)
            pltpu.sync_copy(outf.at[pl.ds(0, W_COLS)],
                            rs_hbm.at[pl.ds(f * VOCAB + cbase, W_COLS)])
            return carry

        lax.fori_loop(0, N_SPARSE, f_body, 0)

    @pl.when(wid == 31)
    def _():
        fire, drain = mk_ring(T31_BASE, SUBC31)
        for sub in range(4):
            fire(0, sub)

        def f_body(f, carry):
            for sub in range(4):
                drain(f, sub)
                reduce_cols(blks[sub], SUBC31, sub * SUBC31)

                @pl.when(f < N_SPARSE - 1)
                def _():
                    fire(f + 1, sub)
            pltpu.sync_copy(
                dm_hbm.at[pl.ds(f * EMB_DIM, EMB_DIM),
                          pl.ds(T31_BASE + 4 * SUBC31, 128)],
                tailb.at[:, pl.ds(0, 128)])
            reduce_cols(tailb, 128, 4 * SUBC31)
            pltpu.sync_copy(outf.at[pl.ds(0, T31_COLS)],
                            rs_hbm.at[pl.ds(f * VOCAB + T31_BASE, T31_COLS)])
            return carry

        lax.fori_loop(0, N_SPARSE, f_body, 0)

    @pl.when(wid < N_SPARSE)
    def _():
        pltpu.sync_copy(tail_hbm.at[pl.ds(wid * 32, 32)], outt)
        pltpu.sync_copy(outt, rs_hbm.at[pl.ds(wid * VOCAB + TAIL_V, 32)])


@functools.partial(
    pl.kernel,
    out_type=jax.ShapeDtypeStruct((NROWS,), jnp.float32),
    mesh=plsc.VectorSubcoreMesh(core_axis_name="c", subcore_axis_name="s"),
    compiler_params=pltpu.CompilerParams(needs_layout_passes=False),
    scratch_types=[
        pltpu.VMEM((EMB_DIM, SUBC31), jnp.float32),
        pltpu.VMEM((EMB_DIM, SUBC31), jnp.float32),
        pltpu.VMEM((EMB_DIM, SUBC31), jnp.float32),
        pltpu.VMEM((EMB_DIM, SUBC31), jnp.float32),
        pltpu.VMEM((T31_COLS,), jnp.float32),
        pltpu.VMEM((EMB_DIM, 128), jnp.float32),
        pltpu.VMEM((32,), jnp.float32),
        pltpu.SemaphoreType.DMA,
        pltpu.SemaphoreType.DMA,
        pltpu.SemaphoreType.DMA,
        pltpu.SemaphoreType.DMA,
    ],
)
def _sc_rowsum(dm_hbm, tail_hbm, rs_hbm, blk0, blk1, blk2, blk3,
               outf, tailb, outt, sem0, sem1, sem2, sem3):
    _rowsum_body(dm_hbm, tail_hbm, rs_hbm, blk0, blk1, blk2, blk3,
                 outf, tailb, outt, sem0, sem1, sem2, sem3)


def kernel(inputs, emb_tables, dense_weight, bias):
    idx = inputs[:, N_DENSE:N_DENSE + N_SPARSE].astype(jnp.int32).reshape(-1)
    dm = jnp.transpose(emb_tables, (0, 2, 1)).reshape(
        N_SPARSE * EMB_DIM, VOCAB)
    tail = jnp.sum(emb_tables[:, TAIL_V:, :], axis=2).reshape(-1)
    rowsum = _sc_rowsum(dm, tail)
    offs = ((jnp.arange(OFF_PERIOD, dtype=jnp.int32) % N_SPARSE)
            * jnp.int32(VOCAB))
    dvt = jnp.concatenate(
        [inputs[:, :N_DENSE],
         jnp.ones((B, 1), jnp.float32),
         jnp.zeros((B, EMB_DIM - N_DENSE - 1), jnp.float32)], axis=1).T
    dwt = jnp.tile(
        jnp.concatenate([dense_weight[:, 0], bias,
                         jnp.zeros((EMB_DIM - N_DENSE - 1,), jnp.float32)]
                        )[:, None], (1, EMB_DIM))
    out = _sc_linear(rowsum, idx, offs, dvt, dwt)
    return out.reshape(B, 1)
